# baseline (speedup 1.0000x reference)
import functools

import jax
import jax.numpy as jnp
from jax import lax
from jax.experimental import pallas as pl
from jax.experimental.pallas import tpu as pltpu
from jax.experimental.pallas import tpu_sc as plsc

B = 4096
K = 20
D = 16
S = K + 1

NC = 2
NS = 16
NW = NC * NS
BW = B // NW


def _make_gather_sc():
    mesh = plsc.VectorSubcoreMesh(core_axis_name="c", subcore_axis_name="s")

    @functools.partial(
        pl.kernel,
        mesh=mesh,
        out_type=[
            jax.ShapeDtypeStruct((NW, BW, D), jnp.float32),
            jax.ShapeDtypeStruct((NW, S, BW, D), jnp.float32),
        ],
        scratch_types=[
            pltpu.VMEM((BW,), jnp.int32),
            pltpu.VMEM((S, BW), jnp.int32),
            pltpu.VMEM((BW, D), jnp.float32),
            pltpu.VMEM((S, BW, D), jnp.float32),
            pltpu.SemaphoreType.DMA,
            pltpu.SemaphoreType.DMA,
        ],
        compiler_params=pltpu.CompilerParams(use_tc_tiling_on_sc=False),
    )
    def _gather_sc(ih_hbm, hu_hbm, ii_hbm, hui_hbm, ie_out, hu_out,
                   ii_v, hui_v, ie_v, hu_v, sem_a, sem_b):
        w = lax.axis_index("s") * NC + lax.axis_index("c")
        pltpu.sync_copy(ii_hbm.at[w], ii_v)
        pltpu.sync_copy(hui_hbm.at[w], hui_v)
        cp_ie = pltpu.async_copy(ih_hbm.at[ii_v], ie_v, sem_a)
        cps = [
            pltpu.async_copy(hu_hbm.at[hui_v.at[j]], hu_v.at[j], sem_b)
            for j in range(S)
        ]
        cp_ie.wait()
        for c in cps:
            c.wait()
        pltpu.sync_copy(ie_v, ie_out.at[w])
        pltpu.sync_copy(hu_v, hu_out.at[w])

    return _gather_sc


def _score_body(ie_ref, hu_ref, out_ref):
    ie = ie_ref[...]
    hu = hu_ref[...].reshape(NW, S, BW * D)
    pe = hu[:, 0, :]
    score_sum = jnp.sum(pe * ie)
    ne = hu[:, 1:, :].reshape(NW * K, BW * D)
    ie_rep = jnp.broadcast_to(ie[:, None, :], (NW, K, BW * D))
    prod = ne * ie_rep.reshape(NW * K, BW * D)
    row = lax.broadcasted_iota(jnp.int32, (BW * D, BW), 0)
    col = lax.broadcasted_iota(jnp.int32, (BW * D, BW), 1)
    m = jnp.where(row // D == col, 1.0, 0.0).astype(jnp.float32)
    norm = jnp.dot(prod, m, preferred_element_type=jnp.float32)
    denom = jnp.sum(jnp.exp(norm).reshape(NW, K, BW), axis=1)
    nll = (jnp.sum(jnp.log(denom)) - score_sum) / B
    out_ref[0, 0] = nll


def kernel(inputs, predict, normal, I_H, H_U):
    ii = inputs.reshape(NW, BW).astype(jnp.int32)
    hui = jnp.concatenate(
        [predict.reshape(B, 1), normal.reshape(B, K)], axis=1
    ).astype(jnp.int32)
    hui = hui.reshape(NW, BW, S).transpose(0, 2, 1)
    ie, hu = _make_gather_sc()(I_H, H_U, ii, hui)
    out = pl.pallas_call(
        _score_body,
        out_shape=jax.ShapeDtypeStruct((1, 1), jnp.float32),
        out_specs=pl.BlockSpec(memory_space=pltpu.SMEM),
    )(ie.reshape(NW, BW * D), hu.reshape(NW * S, BW * D))
    return out.reshape(1)
